# baseline (speedup 1.0000x reference)
import functools

import jax
import jax.numpy as jnp
from jax import lax
from jax.experimental import pallas as pl
from jax.experimental.pallas import tpu as pltpu
from jax.experimental.pallas import tpu_sc as plsc

B = 16384
DIM = 16
N_NEG = 5
NW = 32
BPW = B // NW
NG = BPW // 16

VOC = 1_000_000
VMAIN = 999_936
NCOL = VMAIN // 128
TAILBASE = NCOL * 2048
NBUF = 8
KMAX = 240
NGRP = KMAX // NBUF
NWORDS = (NCOL + 1) * 2048

_mesh = plsc.VectorSubcoreMesh(core_axis_name="c", subcore_axis_name="s")


@functools.partial(
    pl.kernel,
    out_type=(
        jax.ShapeDtypeStruct((NCOL + 1, 16, 128), jnp.float32),
        jax.ShapeDtypeStruct((NCOL + 1, 16, 128), jnp.float32),
    ),
    mesh=_mesh,
    compiler_params=pltpu.CompilerParams(use_tc_tiling_on_sc=True,
                                         needs_layout_passes=False),
    scratch_types=[
        [pltpu.VMEM((16, 128), jnp.float32) for _ in range(NBUF)],
        [pltpu.VMEM((16, 128), jnp.float32) for _ in range(NBUF)],
        [pltpu.SemaphoreType.DMA for _ in range(NBUF)],
        [pltpu.SemaphoreType.DMA for _ in range(NBUF)],
    ],
)
def _reorder(ut_hbm, vt_hbm, utail_hbm, vtail_hbm, wu_hbm, wv_hbm,
             inb, outb, sin, sout):
    wid = lax.axis_index("s") * 2 + lax.axis_index("c")

    def run_table(src, dst):
        def issue(b, c):
            pltpu.async_copy(src.at[:, pl.ds(c * 128, 128)], dst.at[c], sin[b])

        for b in range(NBUF):
            issue(b, b * 32 + wid)

        def body(g, _):
            for b in range(NBUF):
                c = (g * NBUF + b) * 32 + wid
                pltpu.make_async_copy(src.at[:, pl.ds(c * 128, 128)],
                                      dst.at[0], sin[b]).wait()

                @pl.when(g < NGRP - 1)
                def _():
                    issue(b, ((g + 1) * NBUF + b) * 32 + wid)
            return 0

        lax.fori_loop(0, NGRP, body, 0)

        for k in range(5):
            c_w = KMAX * 32 + k * 32
            @pl.when(c_w + wid < NCOL)
            def _(c_w=c_w):
                c = c_w + wid
                pltpu.sync_copy(src.at[:, pl.ds(c * 128, 128)], dst.at[c])

    run_table(ut_hbm, wu_hbm)
    run_table(vt_hbm, wv_hbm)

    @pl.when(wid == 4)
    def _():
        pltpu.sync_copy(utail_hbm, wu_hbm.at[NCOL].at[pl.ds(0, 8)])

    @pl.when(wid == 5)
    def _():
        pltpu.sync_copy(vtail_hbm, wv_hbm.at[NCOL].at[pl.ds(0, 8)])


@functools.partial(
    pl.kernel,
    out_type=(
        jax.ShapeDtypeStruct((B,), jnp.float32),
        jax.ShapeDtypeStruct((B,), jnp.float32),
    ),
    mesh=_mesh,
    compiler_params=pltpu.CompilerParams(use_tc_tiling_on_sc=False),
    scratch_types=[
        pltpu.VMEM((NG // 8, 128), jnp.int32),
        pltpu.VMEM((NG // 8, 128), jnp.int32),
        pltpu.VMEM((N_NEG, NG // 8, 128), jnp.int32),
        pltpu.VMEM((2 * NG, 128), jnp.int32),
        pltpu.VMEM((2 * NG, 128), jnp.int32),
        pltpu.VMEM((10 * NG, 128), jnp.int32),
        pltpu.VMEM((BPW * 16,), jnp.float32),
        pltpu.VMEM((BPW * 16,), jnp.float32),
        pltpu.VMEM((BPW * N_NEG * 16,), jnp.float32),
        pltpu.VMEM((BPW,), jnp.float32),
        pltpu.VMEM((BPW,), jnp.float32),
        pltpu.SemaphoreType.DMA,
    ],
)
def _sc_score(up_hbm, vp_hbm, vn_hbm, wu_hbm, wv_hbm, score_hbm, negd_hbm,
              idx_u, idx_v, idx_n, wxu, wxv, wxn, ubuf, vbuf, nbuf,
              sbuf, qbuf, sem):
    wid = lax.axis_index("s") * 2 + lax.axis_index("c")
    base = wid * BPW

    pltpu.sync_copy(up_hbm.at[wid], idx_u)
    pltpu.sync_copy(vp_hbm.at[wid], idx_v)
    pltpu.sync_copy(vn_hbm.at[wid], idx_n)

    def word_base(idxv):
        is_tail = idxv >= VMAIN
        main = ((idxv >> 7) << 11) + (idxv & 127)
        tail = TAILBASE + ((idxv - VMAIN) << 4)
        w0 = jnp.where(is_tail, tail, main)
        step = jnp.where(is_tail, jnp.full((16,), 1, jnp.int32),
                         jnp.full((16,), 128, jnp.int32))
        return w0, step

    def gen_widx(g, chunk, dst_ref, row0):
        w, step = word_base(chunk)
        for k in range(16):
            dst_ref[row0 + k // 8, pl.ds((k % 8) * 16, 16)] = w
            w = w + step

    def genall(g, _):
        gen_widx(g, idx_u[g // 8, pl.ds((g % 8) * 16, 16)], wxu, 2 * g)
        gen_widx(g, idx_v[g // 8, pl.ds((g % 8) * 16, 16)], wxv, 2 * g)
        for n in range(5):
            gen_widx(g, idx_n[n, g // 8, pl.ds((g % 8) * 16, 16)],
                     wxn, 10 * g + 2 * n)
        return 0

    lax.fori_loop(0, NG, genall, 0)

    copies = []
    for j in range(2 * NG):
        copies.append(pltpu.async_copy(
            wu_hbm.at[wxu.at[j]], ubuf.at[pl.ds(j * 128, 128)], sem))
    for j in range(2 * NG):
        copies.append(pltpu.async_copy(
            wv_hbm.at[wxv.at[j]], vbuf.at[pl.ds(j * 128, 128)], sem))
    for j in range(10 * NG):
        copies.append(pltpu.async_copy(
            wv_hbm.at[wxn.at[j]], nbuf.at[pl.ds(j * 128, 128)], sem))
    for cp in copies:
        cp.wait()

    def dots(g, _):
        u = [ubuf[pl.ds(g * 256 + k * 16, 16)] for k in range(16)]
        acc = u[0] * vbuf[pl.ds(g * 256, 16)]
        for k in range(1, 16):
            acc = acc + u[k] * vbuf[pl.ds(g * 256 + k * 16, 16)]
        sbuf[pl.ds(g * 16, 16)] = acc

        qacc = None
        for k in range(16):
            ns = nbuf[pl.ds(g * 1280 + k * 16, 16)]
            for n in range(1, 5):
                ns = ns + nbuf[pl.ds(g * 1280 + n * 256 + k * 16, 16)]
            qacc = u[k] * ns if qacc is None else qacc + u[k] * ns
        qbuf[pl.ds(g * 16, 16)] = qacc
        return 0

    lax.fori_loop(0, NG, dots, 0)

    pltpu.sync_copy(sbuf, score_hbm.at[pl.ds(base, BPW)])
    pltpu.sync_copy(qbuf, negd_hbm.at[pl.ds(base, BPW)])


def _tc_body(s_ref, q_ref, o_ref):
    def logsig(x):
        return jnp.minimum(x, 0.0) - jnp.log1p(jnp.exp(-jnp.abs(x)))

    total = jnp.sum(logsig(s_ref[...])) + jnp.sum(logsig(-q_ref[...]))
    o_ref[0, 0] = -total / B


def kernel(u_pos, v_pos, v_neg, batch_size, U, V, cluster_means):
    del batch_size, cluster_means
    wu3, wv3 = _reorder(U.T, V.T,
                        U[VMAIN:].reshape(8, 128), V[VMAIN:].reshape(8, 128))
    wu = wu3.reshape(NWORDS)
    wv = wv3.reshape(NWORDS)

    up = u_pos.astype(jnp.int32).reshape(NW, NG // 8, 128)
    vp = v_pos.astype(jnp.int32).reshape(NW, NG // 8, 128)
    vn = (v_neg.astype(jnp.int32).T.reshape(N_NEG, NW, NG // 8, 128)
          .transpose(1, 0, 2, 3))
    score, negd = _sc_score(up, vp, vn, wu, wv)

    out = pl.pallas_call(
        _tc_body,
        out_shape=jax.ShapeDtypeStruct((1, 1), jnp.float32),
        out_specs=pl.BlockSpec(memory_space=pltpu.SMEM),
    )(score.reshape(128, 128), negd.reshape(128, 128))
    return out[0, 0]


# speedup vs baseline: 17.4497x; 17.4497x over previous
import functools

import jax
import jax.numpy as jnp
from jax import lax
from jax.experimental import pallas as pl
from jax.experimental.pallas import tpu as pltpu
from jax.experimental.pallas import tpu_sc as plsc

B = 16384
DIM = 16
N_NEG = 5
NW = 32
BPW = B // NW
NG = BPW // 16

VOC = 1_000_000
VMAIN = 999_936
NCOL = VMAIN // 128
TAILBASE = NCOL * 2048
NBUF = 12
KMAX = 240
NGRP = KMAX // NBUF
NWORDS = (NCOL + 1) * 2048

_mesh = plsc.VectorSubcoreMesh(core_axis_name="c", subcore_axis_name="s")


@functools.partial(
    pl.kernel,
    out_type=(
        jax.ShapeDtypeStruct((NCOL + 1, 16, 128), jnp.float32),
        jax.ShapeDtypeStruct((NCOL + 1, 16, 128), jnp.float32),
    ),
    mesh=_mesh,
    compiler_params=pltpu.CompilerParams(use_tc_tiling_on_sc=True,
                                         needs_layout_passes=False),
    scratch_types=[
        [pltpu.VMEM((16, 128), jnp.float32) for _ in range(NBUF)],
        [pltpu.VMEM((16, 128), jnp.float32) for _ in range(NBUF)],
        [pltpu.SemaphoreType.DMA for _ in range(NBUF)],
        [pltpu.SemaphoreType.DMA for _ in range(NBUF)],
    ],
)
def _reorder(ut_hbm, vt_hbm, utail_hbm, vtail_hbm, wu_hbm, wv_hbm,
             inb, outb, sin, sout):
    wid = lax.axis_index("s") * 2 + lax.axis_index("c")

    def copy_buf(b):
        for d in range(16):
            for s in range(8):
                outb[b][d, pl.ds(s * 16, 16)] = inb[b][d, pl.ds(s * 16, 16)]

    def run_table(src, dst):
        def issue_in(b, c):
            pltpu.async_copy(src.at[:, pl.ds(c * 128, 128)], inb[b], sin[b])

        for b in range(NBUF):
            issue_in(b, b * 32 + wid)

        def body(g, _):
            for b in range(NBUF):
                c = (g * NBUF + b) * 32 + wid
                pltpu.make_async_copy(src.at[:, pl.ds(c * 128, 128)],
                                      inb[b], sin[b]).wait()

                @pl.when(g > 0)
                def _():
                    pltpu.make_async_copy(outb[b], dst.at[0], sout[b]).wait()

                copy_buf(b)
                pltpu.async_copy(outb[b], dst.at[c], sout[b])

                @pl.when(g < NGRP - 1)
                def _():
                    issue_in(b, ((g + 1) * NBUF + b) * 32 + wid)
            return 0

        lax.fori_loop(0, NGRP, body, 0)
        for b in range(NBUF):
            pltpu.make_async_copy(outb[b], dst.at[0], sout[b]).wait()

        for k in range(5):
            c_w = KMAX * 32 + k * 32
            @pl.when(c_w + wid < NCOL)
            def _(c_w=c_w):
                c = c_w + wid
                pltpu.sync_copy(src.at[:, pl.ds(c * 128, 128)], inb[0])
                pltpu.sync_copy(inb[0], dst.at[c])

    run_table(ut_hbm, wu_hbm)
    run_table(vt_hbm, wv_hbm)

    @pl.when(wid == 4)
    def _():
        pltpu.sync_copy(utail_hbm, wu_hbm.at[NCOL].at[pl.ds(0, 8)])

    @pl.when(wid == 5)
    def _():
        pltpu.sync_copy(vtail_hbm, wv_hbm.at[NCOL].at[pl.ds(0, 8)])


@functools.partial(
    pl.kernel,
    out_type=(
        jax.ShapeDtypeStruct((B,), jnp.float32),
        jax.ShapeDtypeStruct((B,), jnp.float32),
    ),
    mesh=_mesh,
    compiler_params=pltpu.CompilerParams(use_tc_tiling_on_sc=False),
    scratch_types=[
        pltpu.VMEM((NG // 8, 128), jnp.int32),
        pltpu.VMEM((NG // 8, 128), jnp.int32),
        pltpu.VMEM((N_NEG, NG // 8, 128), jnp.int32),
        pltpu.VMEM((2 * NG, 128), jnp.int32),
        pltpu.VMEM((2 * NG, 128), jnp.int32),
        pltpu.VMEM((10 * NG, 128), jnp.int32),
        pltpu.VMEM((BPW * 16,), jnp.float32),
        pltpu.VMEM((BPW * 16,), jnp.float32),
        pltpu.VMEM((BPW * N_NEG * 16,), jnp.float32),
        pltpu.VMEM((BPW,), jnp.float32),
        pltpu.VMEM((BPW,), jnp.float32),
        pltpu.SemaphoreType.DMA,
    ],
)
def _sc_score(up_hbm, vp_hbm, vn_hbm, wu_hbm, wv_hbm, score_hbm, negd_hbm,
              idx_u, idx_v, idx_n, wxu, wxv, wxn, ubuf, vbuf, nbuf,
              sbuf, qbuf, sem):
    wid = lax.axis_index("s") * 2 + lax.axis_index("c")
    base = wid * BPW

    pltpu.sync_copy(up_hbm.at[wid], idx_u)
    pltpu.sync_copy(vp_hbm.at[wid], idx_v)
    pltpu.sync_copy(vn_hbm.at[wid], idx_n)

    def word_base(idxv):
        is_tail = idxv >= VMAIN
        main = ((idxv >> 7) << 11) + (idxv & 127)
        tail = TAILBASE + ((idxv - VMAIN) << 4)
        w0 = jnp.where(is_tail, tail, main)
        step = jnp.where(is_tail, jnp.full((16,), 1, jnp.int32),
                         jnp.full((16,), 128, jnp.int32))
        return w0, step

    def gen_widx(g, chunk, dst_ref, row0):
        w, step = word_base(chunk)
        for k in range(16):
            dst_ref[row0 + k // 8, pl.ds((k % 8) * 16, 16)] = w
            w = w + step

    def gen_uv(g, _):
        gen_widx(g, idx_u[g // 8, pl.ds((g % 8) * 16, 16)], wxu, 2 * g)
        gen_widx(g, idx_v[g // 8, pl.ds((g % 8) * 16, 16)], wxv, 2 * g)
        return 0

    lax.fori_loop(0, NG, gen_uv, 0)

    copies = []
    for j in range(2 * NG):
        copies.append(pltpu.async_copy(
            wu_hbm.at[wxu.at[j]], ubuf.at[pl.ds(j * 128, 128)], sem))
    for j in range(2 * NG):
        copies.append(pltpu.async_copy(
            wv_hbm.at[wxv.at[j]], vbuf.at[pl.ds(j * 128, 128)], sem))

    def gen_neg(g, _):
        for n in range(5):
            gen_widx(g, idx_n[n, g // 8, pl.ds((g % 8) * 16, 16)],
                     wxn, 10 * g + 2 * n)
        return 0

    lax.fori_loop(0, NG, gen_neg, 0)

    for j in range(10 * NG):
        copies.append(pltpu.async_copy(
            wv_hbm.at[wxn.at[j]], nbuf.at[pl.ds(j * 128, 128)], sem))
    for cp in copies:
        cp.wait()

    def dots(g, _):
        u = [ubuf[pl.ds(g * 256 + k * 16, 16)] for k in range(16)]
        acc = u[0] * vbuf[pl.ds(g * 256, 16)]
        for k in range(1, 16):
            acc = acc + u[k] * vbuf[pl.ds(g * 256 + k * 16, 16)]
        sbuf[pl.ds(g * 16, 16)] = acc

        qacc = None
        for k in range(16):
            ns = nbuf[pl.ds(g * 1280 + k * 16, 16)]
            for n in range(1, 5):
                ns = ns + nbuf[pl.ds(g * 1280 + n * 256 + k * 16, 16)]
            qacc = u[k] * ns if qacc is None else qacc + u[k] * ns
        qbuf[pl.ds(g * 16, 16)] = qacc
        return 0

    lax.fori_loop(0, NG, dots, 0)

    pltpu.sync_copy(sbuf, score_hbm.at[pl.ds(base, BPW)])
    pltpu.sync_copy(qbuf, negd_hbm.at[pl.ds(base, BPW)])


def _tc_body(s_ref, q_ref, o_ref):
    def logsig(x):
        return jnp.minimum(x, 0.0) - jnp.log1p(jnp.exp(-jnp.abs(x)))

    total = jnp.sum(logsig(s_ref[...])) + jnp.sum(logsig(-q_ref[...]))
    o_ref[0, 0] = -total / B


def kernel(u_pos, v_pos, v_neg, batch_size, U, V, cluster_means):
    del batch_size, cluster_means
    wu3, wv3 = _reorder(U.T, V.T,
                        U[VMAIN:].reshape(8, 128), V[VMAIN:].reshape(8, 128))
    wu = wu3.reshape(NWORDS)
    wv = wv3.reshape(NWORDS)

    up = u_pos.astype(jnp.int32).reshape(NW, NG // 8, 128)
    vp = v_pos.astype(jnp.int32).reshape(NW, NG // 8, 128)
    vn = (v_neg.astype(jnp.int32).T.reshape(N_NEG, NW, NG // 8, 128)
          .transpose(1, 0, 2, 3))
    score, negd = _sc_score(up, vp, vn, wu, wv)

    out = pl.pallas_call(
        _tc_body,
        out_shape=jax.ShapeDtypeStruct((1, 1), jnp.float32),
        out_specs=pl.BlockSpec(memory_space=pltpu.SMEM),
    )(score.reshape(128, 128), negd.reshape(128, 128))
    return out[0, 0]


# speedup vs baseline: 19.1341x; 1.0965x over previous
import functools

import jax
import jax.numpy as jnp
from jax import lax
from jax.experimental import pallas as pl
from jax.experimental.pallas import tpu as pltpu
from jax.experimental.pallas import tpu_sc as plsc

B = 16384
DIM = 16
N_NEG = 5
NW = 32
BPW = B // NW
NG = BPW // 16

VOC = 1_000_000
VMAIN = 999_936
NCOL = VMAIN // 128
TAILBASE = NCOL * 2048
NBUF = 8
KMAX = 240
NGRP = KMAX // NBUF
NWORDS = (NCOL + 1) * 2048

_mesh = plsc.VectorSubcoreMesh(core_axis_name="c", subcore_axis_name="s")


@functools.partial(
    pl.kernel,
    out_type=(
        jax.ShapeDtypeStruct((NCOL + 1, 16, 128), jnp.float32),
        jax.ShapeDtypeStruct((NCOL + 1, 16, 128), jnp.float32),
    ),
    mesh=_mesh,
    compiler_params=pltpu.CompilerParams(use_tc_tiling_on_sc=True,
                                         needs_layout_passes=False),
    scratch_types=[
        [pltpu.VMEM((16, 128), jnp.float32) for _ in range(NBUF)],
        [pltpu.VMEM((16, 128), jnp.float32) for _ in range(NBUF)],
        [pltpu.SemaphoreType.DMA for _ in range(NBUF)],
        [pltpu.SemaphoreType.DMA for _ in range(NBUF)],
    ],
)
def _reorder(ut_hbm, vt_hbm, utail_hbm, vtail_hbm, wu_hbm, wv_hbm,
             inb, outb, sin, sout):
    wid = lax.axis_index("s") * 2 + lax.axis_index("c")

    def copy_buf(b):
        for d in range(16):
            for s in range(8):
                outb[b][d, pl.ds(s * 16, 16)] = inb[b][d, pl.ds(s * 16, 16)]

    def run_table(src, dst):
        def issue_in(b, c):
            pltpu.async_copy(src.at[:, pl.ds(c * 128, 128)], inb[b], sin[b])

        for b in range(NBUF):
            issue_in(b, b * 32 + wid)

        def body(g, _):
            for b in range(NBUF):
                c = (g * NBUF + b) * 32 + wid
                pltpu.make_async_copy(src.at[:, pl.ds(c * 128, 128)],
                                      inb[b], sin[b]).wait()

                @pl.when(g > 0)
                def _():
                    pltpu.make_async_copy(outb[b], dst.at[0], sout[b]).wait()

                copy_buf(b)
                pltpu.async_copy(outb[b], dst.at[c], sout[b])

                @pl.when(g < NGRP - 1)
                def _():
                    issue_in(b, ((g + 1) * NBUF + b) * 32 + wid)
            return 0

        lax.fori_loop(0, NGRP, body, 0)
        for b in range(NBUF):
            pltpu.make_async_copy(outb[b], dst.at[0], sout[b]).wait()

        for k in range(5):
            c_w = KMAX * 32 + k * 32
            @pl.when(c_w + wid < NCOL)
            def _(c_w=c_w):
                c = c_w + wid
                pltpu.sync_copy(src.at[:, pl.ds(c * 128, 128)], inb[0])
                pltpu.sync_copy(inb[0], dst.at[c])

    run_table(ut_hbm, wu_hbm)
    run_table(vt_hbm, wv_hbm)

    @pl.when(wid == 4)
    def _():
        pltpu.sync_copy(utail_hbm, wu_hbm.at[NCOL].at[pl.ds(0, 8)])

    @pl.when(wid == 5)
    def _():
        pltpu.sync_copy(vtail_hbm, wv_hbm.at[NCOL].at[pl.ds(0, 8)])


@functools.partial(
    pl.kernel,
    out_type=(
        jax.ShapeDtypeStruct((B,), jnp.float32),
        jax.ShapeDtypeStruct((B,), jnp.float32),
    ),
    mesh=_mesh,
    compiler_params=pltpu.CompilerParams(use_tc_tiling_on_sc=False),
    scratch_types=[
        pltpu.VMEM((NG // 8, 128), jnp.int32),
        pltpu.VMEM((NG // 8, 128), jnp.int32),
        pltpu.VMEM((N_NEG, NG // 8, 128), jnp.int32),
        pltpu.VMEM((2 * NG, 128), jnp.int32),
        pltpu.VMEM((2 * NG, 128), jnp.int32),
        pltpu.VMEM((10 * NG, 128), jnp.int32),
        pltpu.VMEM((BPW * 16,), jnp.float32),
        pltpu.VMEM((BPW * 16,), jnp.float32),
        pltpu.VMEM((BPW * N_NEG * 16,), jnp.float32),
        pltpu.VMEM((BPW,), jnp.float32),
        pltpu.VMEM((BPW,), jnp.float32),
        pltpu.SemaphoreType.DMA,
    ],
)
def _sc_score(up_hbm, vp_hbm, vn_hbm, wu_hbm, wv_hbm, score_hbm, negd_hbm,
              idx_u, idx_v, idx_n, wxu, wxv, wxn, ubuf, vbuf, nbuf,
              sbuf, qbuf, sem):
    wid = lax.axis_index("s") * 2 + lax.axis_index("c")
    base = wid * BPW

    pltpu.sync_copy(up_hbm.at[wid], idx_u)
    pltpu.sync_copy(vp_hbm.at[wid], idx_v)
    pltpu.sync_copy(vn_hbm.at[wid], idx_n)

    def word_base(idxv):
        is_tail = idxv >= VMAIN
        main = ((idxv >> 7) << 11) + (idxv & 127)
        tail = TAILBASE + ((idxv - VMAIN) << 4)
        w0 = jnp.where(is_tail, tail, main)
        step = jnp.where(is_tail, jnp.full((16,), 1, jnp.int32),
                         jnp.full((16,), 128, jnp.int32))
        return w0, step

    def gen_widx(g, chunk, dst_ref, row0):
        w, step = word_base(chunk)
        for k in range(16):
            dst_ref[row0 + k // 8, pl.ds((k % 8) * 16, 16)] = w
            w = w + step

    def genall(g, _):
        gen_widx(g, idx_u[g // 8, pl.ds((g % 8) * 16, 16)], wxu, 2 * g)
        gen_widx(g, idx_v[g // 8, pl.ds((g % 8) * 16, 16)], wxv, 2 * g)
        for n in range(5):
            gen_widx(g, idx_n[n, g // 8, pl.ds((g % 8) * 16, 16)],
                     wxn, 10 * g + 2 * n)
        return 0

    lax.fori_loop(0, NG, genall, 0)

    copies = []
    for j in range(2 * NG):
        copies.append(pltpu.async_copy(
            wu_hbm.at[wxu.at[j]], ubuf.at[pl.ds(j * 128, 128)], sem))
    for j in range(2 * NG):
        copies.append(pltpu.async_copy(
            wv_hbm.at[wxv.at[j]], vbuf.at[pl.ds(j * 128, 128)], sem))
    for j in range(10 * NG):
        copies.append(pltpu.async_copy(
            wv_hbm.at[wxn.at[j]], nbuf.at[pl.ds(j * 128, 128)], sem))
    for cp in copies:
        cp.wait()

    def dots(g, _):
        u = [ubuf[pl.ds(g * 256 + k * 16, 16)] for k in range(16)]
        acc = u[0] * vbuf[pl.ds(g * 256, 16)]
        for k in range(1, 16):
            acc = acc + u[k] * vbuf[pl.ds(g * 256 + k * 16, 16)]
        sbuf[pl.ds(g * 16, 16)] = acc

        qacc = None
        for k in range(16):
            ns = nbuf[pl.ds(g * 1280 + k * 16, 16)]
            for n in range(1, 5):
                ns = ns + nbuf[pl.ds(g * 1280 + n * 256 + k * 16, 16)]
            qacc = u[k] * ns if qacc is None else qacc + u[k] * ns
        qbuf[pl.ds(g * 16, 16)] = qacc
        return 0

    lax.fori_loop(0, NG, dots, 0)

    pltpu.sync_copy(sbuf, score_hbm.at[pl.ds(base, BPW)])
    pltpu.sync_copy(qbuf, negd_hbm.at[pl.ds(base, BPW)])


def _tc_body(s_ref, q_ref, o_ref):
    def logsig(x):
        return jnp.minimum(x, 0.0) - jnp.log1p(jnp.exp(-jnp.abs(x)))

    total = jnp.sum(logsig(s_ref[...])) + jnp.sum(logsig(-q_ref[...]))
    o_ref[0, 0] = -total / B


def kernel(u_pos, v_pos, v_neg, batch_size, U, V, cluster_means):
    del batch_size, cluster_means
    wu3, wv3 = _reorder(U.T, V.T,
                        U[VMAIN:].reshape(8, 128), V[VMAIN:].reshape(8, 128))
    wu = wu3.reshape(NWORDS)
    wv = wv3.reshape(NWORDS)

    up = u_pos.astype(jnp.int32).reshape(NW, NG // 8, 128)
    vp = v_pos.astype(jnp.int32).reshape(NW, NG // 8, 128)
    vn = (v_neg.astype(jnp.int32).T.reshape(N_NEG, NW, NG // 8, 128)
          .transpose(1, 0, 2, 3))
    score, negd = _sc_score(up, vp, vn, wu, wv)

    out = pl.pallas_call(
        _tc_body,
        out_shape=jax.ShapeDtypeStruct((1, 1), jnp.float32),
        out_specs=pl.BlockSpec(memory_space=pltpu.SMEM),
    )(score.reshape(128, 128), negd.reshape(128, 128))
    return out[0, 0]


# speedup vs baseline: 19.3219x; 1.0098x over previous
import functools

import jax
import jax.numpy as jnp
from jax import lax
from jax.experimental import pallas as pl
from jax.experimental.pallas import tpu as pltpu
from jax.experimental.pallas import tpu_sc as plsc

B = 16384
DIM = 16
N_NEG = 5
NW = 32
BPW = B // NW
NG = BPW // 16

VOC = 1_000_000
VMAIN = 999_936
NCOL = VMAIN // 128
TAILBASE = NCOL * 2048
NBUF = 8
KMAX = 240
NGRP = KMAX // NBUF
NWORDS = (NCOL + 1) * 2048

_mesh = plsc.VectorSubcoreMesh(core_axis_name="c", subcore_axis_name="s")


@functools.partial(
    pl.kernel,
    out_type=(
        jax.ShapeDtypeStruct((NCOL + 1, 16, 128), jnp.float32),
        jax.ShapeDtypeStruct((NCOL + 1, 16, 128), jnp.float32),
    ),
    mesh=_mesh,
    compiler_params=pltpu.CompilerParams(use_tc_tiling_on_sc=True,
                                         needs_layout_passes=False),
    scratch_types=[
        [pltpu.VMEM((16, 128), jnp.float32) for _ in range(NBUF)],
        [pltpu.VMEM((16, 128), jnp.float32) for _ in range(NBUF)],
        [pltpu.SemaphoreType.DMA for _ in range(NBUF)],
        [pltpu.SemaphoreType.DMA for _ in range(NBUF)],
    ],
)
def _reorder(ut_hbm, vt_hbm, utail_hbm, vtail_hbm, wu_hbm, wv_hbm,
             inb, outb, sin, sout):
    wid = lax.axis_index("s") * 2 + lax.axis_index("c")

    def copy_buf(b):
        for d in range(16):
            for s in range(8):
                outb[b][d, pl.ds(s * 16, 16)] = inb[b][d, pl.ds(s * 16, 16)]

    def run_table(src, dst):
        def issue_in(b, c):
            pltpu.async_copy(src.at[:, pl.ds(c * 128, 128)], inb[b], sin[b])

        for b in range(NBUF):
            issue_in(b, b * 32 + wid)

        def body(g, _):
            for b in range(NBUF):
                c = (g * NBUF + b) * 32 + wid
                pltpu.make_async_copy(src.at[:, pl.ds(c * 128, 128)],
                                      inb[b], sin[b]).wait()

                @pl.when(g > 0)
                def _():
                    pltpu.make_async_copy(outb[b], dst.at[0], sout[b]).wait()

                copy_buf(b)
                pltpu.async_copy(outb[b], dst.at[c], sout[b])

                @pl.when(g < NGRP - 1)
                def _():
                    issue_in(b, ((g + 1) * NBUF + b) * 32 + wid)
            return 0

        lax.fori_loop(0, NGRP, body, 0)
        for b in range(NBUF):
            pltpu.make_async_copy(outb[b], dst.at[0], sout[b]).wait()

        for k in range(5):
            c_w = KMAX * 32 + k * 32
            @pl.when(c_w + wid < NCOL)
            def _(c_w=c_w):
                c = c_w + wid
                pltpu.sync_copy(src.at[:, pl.ds(c * 128, 128)], inb[0])
                pltpu.sync_copy(inb[0], dst.at[c])

    run_table(ut_hbm, wu_hbm)
    run_table(vt_hbm, wv_hbm)

    @pl.when(wid == 4)
    def _():
        pltpu.sync_copy(utail_hbm, wu_hbm.at[NCOL].at[pl.ds(0, 8)])

    @pl.when(wid == 5)
    def _():
        pltpu.sync_copy(vtail_hbm, wv_hbm.at[NCOL].at[pl.ds(0, 8)])


@functools.partial(
    pl.kernel,
    out_type=(
        jax.ShapeDtypeStruct((B,), jnp.float32),
        jax.ShapeDtypeStruct((B,), jnp.float32),
    ),
    mesh=_mesh,
    compiler_params=pltpu.CompilerParams(use_tc_tiling_on_sc=False),
    scratch_types=[
        pltpu.VMEM((NG // 8, 128), jnp.int32),
        pltpu.VMEM((NG // 8, 128), jnp.int32),
        pltpu.VMEM((N_NEG, NG // 8, 128), jnp.int32),
        pltpu.VMEM((2 * NG, 128), jnp.int32),
        pltpu.VMEM((2 * NG, 128), jnp.int32),
        pltpu.VMEM((10 * NG, 128), jnp.int32),
        pltpu.VMEM((BPW * 16,), jnp.float32),
        pltpu.VMEM((BPW * 16,), jnp.float32),
        pltpu.VMEM((BPW * N_NEG * 16,), jnp.float32),
        pltpu.VMEM((BPW,), jnp.float32),
        pltpu.VMEM((BPW,), jnp.float32),
        pltpu.SemaphoreType.DMA,
    ],
)
def _sc_score(up_hbm, vp_hbm, vn_hbm, wu_hbm, wv_hbm, score_hbm, negd_hbm,
              idx_u, idx_v, idx_n, wxu, wxv, wxn, ubuf, vbuf, nbuf,
              sbuf, qbuf, sem):
    wid = lax.axis_index("s") * 2 + lax.axis_index("c")
    base = wid * BPW

    pltpu.sync_copy(up_hbm.at[wid], idx_u)
    pltpu.sync_copy(vp_hbm.at[wid], idx_v)
    pltpu.sync_copy(vn_hbm.at[wid], idx_n)

    def word_base(idxv):
        is_tail = idxv >= VMAIN
        main = ((idxv >> 7) << 11) + (idxv & 127)
        tail = TAILBASE + ((idxv - VMAIN) << 4)
        w0 = jnp.where(is_tail, tail, main)
        step = jnp.where(is_tail, jnp.full((16,), 1, jnp.int32),
                         jnp.full((16,), 128, jnp.int32))
        return w0, step

    def gen_widx(g, chunk, dst_ref, row0):
        w, step = word_base(chunk)
        for k in range(16):
            dst_ref[row0 + k // 8, pl.ds((k % 8) * 16, 16)] = w
            w = w + step

    def gen_uv(g, _):
        gen_widx(g, idx_u[g // 8, pl.ds((g % 8) * 16, 16)], wxu, 2 * g)
        gen_widx(g, idx_v[g // 8, pl.ds((g % 8) * 16, 16)], wxv, 2 * g)
        return 0

    lax.fori_loop(0, NG, gen_uv, 0)

    copies = []
    for j in range(2 * NG):
        copies.append(pltpu.async_copy(
            wu_hbm.at[wxu.at[j]], ubuf.at[pl.ds(j * 128, 128)], sem))
    for j in range(2 * NG):
        copies.append(pltpu.async_copy(
            wv_hbm.at[wxv.at[j]], vbuf.at[pl.ds(j * 128, 128)], sem))

    def gen_neg(g, _):
        for n in range(5):
            gen_widx(g, idx_n[n, g // 8, pl.ds((g % 8) * 16, 16)],
                     wxn, 10 * g + 2 * n)
        return 0

    lax.fori_loop(0, NG, gen_neg, 0)

    for j in range(10 * NG):
        copies.append(pltpu.async_copy(
            wv_hbm.at[wxn.at[j]], nbuf.at[pl.ds(j * 128, 128)], sem))
    for cp in copies:
        cp.wait()

    def dots(g, _):
        u = [ubuf[pl.ds(g * 256 + k * 16, 16)] for k in range(16)]
        acc = u[0] * vbuf[pl.ds(g * 256, 16)]
        for k in range(1, 16):
            acc = acc + u[k] * vbuf[pl.ds(g * 256 + k * 16, 16)]
        sbuf[pl.ds(g * 16, 16)] = acc

        qacc = None
        for k in range(16):
            ns = nbuf[pl.ds(g * 1280 + k * 16, 16)]
            for n in range(1, 5):
                ns = ns + nbuf[pl.ds(g * 1280 + n * 256 + k * 16, 16)]
            qacc = u[k] * ns if qacc is None else qacc + u[k] * ns
        qbuf[pl.ds(g * 16, 16)] = qacc
        return 0

    lax.fori_loop(0, NG, dots, 0)

    pltpu.sync_copy(sbuf, score_hbm.at[pl.ds(base, BPW)])
    pltpu.sync_copy(qbuf, negd_hbm.at[pl.ds(base, BPW)])


def _tc_body(s_ref, q_ref, o_ref):
    def logsig(x):
        return jnp.minimum(x, 0.0) - jnp.log1p(jnp.exp(-jnp.abs(x)))

    total = jnp.sum(logsig(s_ref[...])) + jnp.sum(logsig(-q_ref[...]))
    o_ref[0, 0] = -total / B


def kernel(u_pos, v_pos, v_neg, batch_size, U, V, cluster_means):
    del batch_size, cluster_means
    wu3, wv3 = _reorder(U.T, V.T,
                        U[VMAIN:].reshape(8, 128), V[VMAIN:].reshape(8, 128))
    wu = wu3.reshape(NWORDS)
    wv = wv3.reshape(NWORDS)

    up = u_pos.astype(jnp.int32).reshape(NW, NG // 8, 128)
    vp = v_pos.astype(jnp.int32).reshape(NW, NG // 8, 128)
    vn = (v_neg.astype(jnp.int32).T.reshape(N_NEG, NW, NG // 8, 128)
          .transpose(1, 0, 2, 3))
    score, negd = _sc_score(up, vp, vn, wu, wv)

    out = pl.pallas_call(
        _tc_body,
        out_shape=jax.ShapeDtypeStruct((1, 1), jnp.float32),
        out_specs=pl.BlockSpec(memory_space=pltpu.SMEM),
    )(score.reshape(128, 128), negd.reshape(128, 128))
    return out[0, 0]
